# baseline (speedup 1.0000x reference)
import functools
import jax
import jax.numpy as jnp
from jax import lax
from jax.experimental import pallas as pl
from jax.experimental.pallas import tpu as pltpu
from jax.experimental.pallas import tpu_sc as plsc

VOCAB = 1000000
EMBED_DIM = 64
BATCH = 16384

NUM_CORES = 2
NUM_SUBCORES = 16
LANES = 16
NUM_WORKERS = NUM_CORES * NUM_SUBCORES
B_PER_W = BATCH // NUM_WORKERS
WIDE = 2 * EMBED_DIM

STRIPE_W = 384
N_STRIPES = -(-VOCAB // STRIPE_W)
LAST_W = VOCAB - (N_STRIPES - 1) * STRIPE_W
STRIPES_PER_W = -(-N_STRIPES // NUM_WORKERS)
ECHUNK = 512
OFFS_PAD = (-(-(N_STRIPES + 2) // LANES) + 1) * LANES


def _scan_body(tbl_hbm, tail_hbm, svals_hbm, sids_hbm, offs_hbm, stage_hbm,
               buf, evals, eids, rowstage, gidx, offs_v, sem, esem):
    wid = lax.axis_index("s") * NUM_CORES + lax.axis_index("c")

    pltpu.sync_copy(offs_hbm, offs_v)

    lane = lax.iota(jnp.int32, LANES)

    def scalar_at(i):
        vbase = (i // LANES) * LANES
        vec = offs_v[pl.ds(vbase, LANES)]
        sel = lax.select(lane == i - vbase, vec,
                         jnp.zeros((LANES,), jnp.int32))
        return jnp.max(sel)

    def fire(t):
        s = wid + t * NUM_WORKERS
        par = lax.rem(t, 2)

        @pl.when(s < N_STRIPES - 1)
        def _():
            colw = pl.multiple_of(s * STRIPE_W, 128)
            for a in range(EMBED_DIM // 8):
                pltpu.async_copy(
                    tbl_hbm.at[pl.ds(a * 8, 8), pl.ds(colw, STRIPE_W)],
                    buf.at[par, pl.ds(a * 8, 8), :], sem.at[par])

        @pl.when(s == N_STRIPES - 1)
        def _():
            pltpu.async_copy(tail_hbm, buf.at[par, :, pl.ds(0, 128)],
                             sem.at[par])

    def drain(t):
        s = wid + t * NUM_WORKERS
        par = lax.rem(t, 2)

        @pl.when(s < N_STRIPES - 1)
        def _():
            for a in range(EMBED_DIM // 8):
                pltpu.make_async_copy(
                    tbl_hbm.at[pl.ds(a * 8, 8), pl.ds(0, STRIPE_W)],
                    buf.at[par, pl.ds(a * 8, 8), :], sem.at[par]).wait()

        @pl.when(s == N_STRIPES - 1)
        def _():
            pltpu.make_async_copy(tail_hbm, buf.at[par, :, pl.ds(0, 128)],
                                  sem.at[par]).wait()

    def do_stripe(s, par, width):
        seg_lo = scalar_at(s)
        seg_hi = scalar_at(s + 1)
        start = pl.multiple_of((seg_lo // 8) * 8, 8)
        count = seg_hi - start
        col0 = s * STRIPE_W
        parv = jnp.full((LANES,), 0, jnp.int32) + par

        def chunk_body(ch, carry):
            ebase = pl.multiple_of(start + ch * ECHUNK, 8)
            pltpu.async_copy(svals_hbm.at[pl.ds(ebase, ECHUNK)], evals,
                             esem).wait()
            pltpu.async_copy(sids_hbm.at[pl.ds(ebase, ECHUNK)], eids,
                             esem).wait()

            def group_body(g, carry2):
                gsl = pl.ds(g * LANES, LANES)
                pos = ebase + g * LANES + lane
                valid = jnp.logical_and(pos >= seg_lo, pos < seg_hi)
                c_loc = evals[gsl] - col0
                c_loc = lax.max(jnp.zeros((LANES,), jnp.int32),
                                lax.min(c_loc,
                                        jnp.full((LANES,), width - 1,
                                                 jnp.int32)))
                ids = lax.select(valid, eids[gsl],
                                 jnp.full((LANES,), -1, jnp.int32))
                gidx[...] = ids
                for j in range(EMBED_DIM):
                    jv = jnp.full((LANES,), j, jnp.int32)
                    vv = plsc.load_gather(buf, [parv, jv, c_loc])
                    plsc.store_scatter(rowstage, [lane, jv], vv)
                pltpu.async_copy(
                    rowstage,
                    stage_hbm.at[plsc.Indices(gidx, ignored_value=-1)],
                    esem).wait()
                return carry2

            in_chunk = lax.min(seg_hi - ebase, ECHUNK)
            ngroups = lax.div(in_chunk + (LANES - 1), LANES)
            lax.fori_loop(0, ngroups, group_body, 0)
            return carry

        nchunks = lax.div(count + (ECHUNK - 1), ECHUNK)
        lax.fori_loop(0, nchunks, chunk_body, 0)

    fire(0)

    def stripe_loop(t, carry):
        s = wid + t * NUM_WORKERS

        @pl.when(t + 1 < STRIPES_PER_W)
        def _():
            fire(t + 1)

        drain(t)
        par = lax.rem(t, 2)

        @pl.when(s < N_STRIPES - 1)
        def _():
            do_stripe(s, par, STRIPE_W)

        @pl.when(s == N_STRIPES - 1)
        def _():
            do_stripe(s, par, LAST_W)

        return carry

    lax.fori_loop(0, STRIPES_PER_W, stripe_loop, 0)


DCHUNK = 128


def _dot_body(vstage_hbm, ustage_hbm, out_hbm, vrows, urows, res_v, sem):
    wid = lax.axis_index("s") * NUM_CORES + lax.axis_index("c")
    base = wid * B_PER_W

    lane = lax.iota(jnp.int32, LANES)

    def chunk_body(k, carry):
        rbase = base + k * DCHUNK
        cp1 = pltpu.async_copy(vstage_hbm.at[pl.ds(rbase, DCHUNK)], vrows,
                               sem)
        cp2 = pltpu.async_copy(ustage_hbm.at[pl.ds(rbase, DCHUNK)], urows,
                               sem)
        cp1.wait()
        cp2.wait()
        for g in range(DCHUNK // LANES):
            slot = g * LANES + lane
            acc = jnp.zeros((LANES,), jnp.float32)
            for j in range(EMBED_DIM):
                jv = jnp.full((LANES,), j, jnp.int32)
                vv = plsc.load_gather(vrows, [slot, jv])
                uu = plsc.load_gather(urows, [slot, jv])
                acc = acc + vv * uu
            res_v[pl.ds(k * DCHUNK + g * LANES, LANES)] = acc
        return carry

    lax.fori_loop(0, B_PER_W // DCHUNK, chunk_body, 0)

    pltpu.sync_copy(res_v, out_hbm.at[pl.ds(base, B_PER_W)])


def _make_mesh():
    return plsc.VectorSubcoreMesh(
        core_axis_name="c", subcore_axis_name="s",
        num_cores=NUM_CORES, num_subcores=NUM_SUBCORES)


def _scan_call(tbl, tail, svals, sids, offs):
    k = pl.kernel(
        _scan_body,
        out_type=jax.ShapeDtypeStruct((BATCH, WIDE), jnp.float32),
        mesh=_make_mesh(),
        compiler_params=pltpu.CompilerParams(needs_layout_passes=False),
        scratch_types=[
            pltpu.VMEM((2, EMBED_DIM, STRIPE_W), jnp.float32),
            pltpu.VMEM((ECHUNK,), jnp.int32),
            pltpu.VMEM((ECHUNK,), jnp.int32),
            pltpu.VMEM((LANES, WIDE), jnp.float32),
            pltpu.VMEM((LANES,), jnp.int32),
            pltpu.VMEM((OFFS_PAD,), jnp.int32),
            pltpu.SemaphoreType.DMA((2,)),
            pltpu.SemaphoreType.DMA,
        ],
    )
    return k(tbl, tail, svals, sids, offs)


def _dot_call(vstage, ustage):
    k = pl.kernel(
        _dot_body,
        out_type=jax.ShapeDtypeStruct((BATCH,), jnp.float32),
        mesh=_make_mesh(),
        compiler_params=pltpu.CompilerParams(needs_layout_passes=False),
        scratch_types=[
            pltpu.VMEM((DCHUNK, WIDE), jnp.float32),
            pltpu.VMEM((DCHUNK, WIDE), jnp.float32),
            pltpu.VMEM((B_PER_W,), jnp.float32),
            pltpu.SemaphoreType.DMA,
        ],
    )
    return k(vstage, ustage)


def _prep(idx):
    idx = idx.astype(jnp.int32)
    stripe = idx // STRIPE_W
    packed = stripe * BATCH + lax.iota(jnp.int32, BATCH)
    packed = jnp.sort(packed)
    sids = packed % BATCH
    svals = jnp.take(idx, sids, axis=0)
    sstripes = packed // BATCH
    bounds = lax.iota(jnp.int32, N_STRIPES + 1)
    offs = jnp.searchsorted(sstripes, bounds, side="left").astype(jnp.int32)
    offs = jnp.pad(offs, (0, OFFS_PAD - offs.shape[0]))
    svals = jnp.pad(svals, (0, ECHUNK))
    sids = jnp.pad(sids, (0, ECHUNK), constant_values=-1)
    return svals, sids, offs


def _tail_view(tbl):
    t = tbl[VOCAB - LAST_W:].T
    return jnp.pad(t, ((0, 0), (0, 128 - LAST_W)))


@jax.jit
def kernel(center_ids, context_ids, in_embed, out_embed):
    csv, csi, cof = _prep(center_ids)
    xsv, xsi, xof = _prep(context_ids)
    tin, ttin = in_embed.T, _tail_view(in_embed)
    tout, ttout = out_embed.T, _tail_view(out_embed)
    vstage = _scan_call(tin, ttin, csv, csi, cof)
    ustage = _scan_call(tout, ttout, xsv, xsi, xof)
    return _dot_call(vstage, ustage)


# speedup vs baseline: 1.1267x; 1.1267x over previous
import functools
import jax
import jax.numpy as jnp
from jax import lax
from jax.experimental import pallas as pl
from jax.experimental.pallas import tpu as pltpu
from jax.experimental.pallas import tpu_sc as plsc

VOCAB = 1000000
EMBED_DIM = 64
BATCH = 16384

NUM_CORES = 2
NUM_SUBCORES = 16
LANES = 16
NUM_WORKERS = NUM_CORES * NUM_SUBCORES
B_PER_W = BATCH // NUM_WORKERS
WIDE = 2 * EMBED_DIM

STRIPE_W = 384
N_STRIPES = -(-VOCAB // STRIPE_W)
LAST_W = VOCAB - (N_STRIPES - 1) * STRIPE_W
STRIPES_PER_W = -(-N_STRIPES // NUM_WORKERS)
ECHUNK = 512
EVCAP = (-(-(BATCH + 8) // ECHUNK)) * ECHUNK
OFFS_PAD = (-(-(N_STRIPES + 2) // LANES) + 1) * LANES


def _scan_body(tbl_hbm, tail_hbm, svals_hbm, sids_hbm, offs_hbm, stage_hbm,
               buf, evals, eids, rowstage, gidx, offs_v, sem, esem):
    wid = lax.axis_index("s") * NUM_CORES + lax.axis_index("c")

    pltpu.sync_copy(offs_hbm, offs_v)

    lane = lax.iota(jnp.int32, LANES)

    def scalar_at(i):
        vbase = (i // LANES) * LANES
        vec = offs_v[pl.ds(vbase, LANES)]
        sel = lax.select(lane == i - vbase, vec,
                         jnp.zeros((LANES,), jnp.int32))
        return jnp.max(sel)

    s0 = wid * STRIPES_PER_W
    send = lax.min(s0 + STRIPES_PER_W, N_STRIPES)
    rng_lo = scalar_at(s0)
    rng_hi = scalar_at(send)
    ebase0 = pl.multiple_of((rng_lo // 8) * 8, 8)

    def eload(i, carry):
        src = pl.multiple_of(ebase0 + i * ECHUNK, 8)
        dst = pl.ds(i * ECHUNK, ECHUNK)
        pltpu.async_copy(svals_hbm.at[pl.ds(src, ECHUNK)], evals.at[dst],
                         esem)
        pltpu.async_copy(sids_hbm.at[pl.ds(src, ECHUNK)], eids.at[dst],
                         esem)
        return carry

    def edrain(i, carry):
        dst = pl.ds(i * ECHUNK, ECHUNK)
        pltpu.make_async_copy(svals_hbm.at[pl.ds(0, ECHUNK)], evals.at[dst],
                              esem).wait()
        pltpu.make_async_copy(sids_hbm.at[pl.ds(0, ECHUNK)], eids.at[dst],
                              esem).wait()
        return carry

    nech = lax.div(rng_hi - ebase0 + (ECHUNK - 1), ECHUNK)
    lax.fori_loop(0, nech, eload, 0)

    def fire(t):
        s = s0 + t
        par = lax.rem(t, 2)

        @pl.when(s < N_STRIPES - 1)
        def _():
            colw = pl.multiple_of(s * STRIPE_W, 128)
            for a in range(EMBED_DIM // 8):
                pltpu.async_copy(
                    tbl_hbm.at[pl.ds(a * 8, 8), pl.ds(colw, STRIPE_W)],
                    buf.at[par, pl.ds(a * 8, 8), :], sem.at[par])

        @pl.when(s == N_STRIPES - 1)
        def _():
            pltpu.async_copy(tail_hbm, buf.at[par, :, pl.ds(0, 128)],
                             sem.at[par])

    def drain(t):
        s = s0 + t
        par = lax.rem(t, 2)

        @pl.when(s < N_STRIPES - 1)
        def _():
            for a in range(EMBED_DIM // 8):
                pltpu.make_async_copy(
                    tbl_hbm.at[pl.ds(a * 8, 8), pl.ds(0, STRIPE_W)],
                    buf.at[par, pl.ds(a * 8, 8), :], sem.at[par]).wait()

        @pl.when(s == N_STRIPES - 1)
        def _():
            pltpu.make_async_copy(tail_hbm, buf.at[par, :, pl.ds(0, 128)],
                                  sem.at[par]).wait()

    def do_stripe(s, par, width):
        seg_lo = scalar_at(s)
        seg_hi = scalar_at(s + 1)
        col0 = s * STRIPE_W
        parv = jnp.full((LANES,), 0, jnp.int32) + par
        p0 = ((seg_lo - ebase0) // LANES) * LANES

        def group_body(g, carry2):
            gsl = pl.ds(p0 + g * LANES, LANES)
            pos = ebase0 + p0 + g * LANES + lane
            valid = jnp.logical_and(pos >= seg_lo, pos < seg_hi)
            c_loc = evals[gsl] - col0
            c_loc = lax.max(jnp.zeros((LANES,), jnp.int32),
                            lax.min(c_loc,
                                    jnp.full((LANES,), width - 1,
                                             jnp.int32)))
            ids = lax.select(valid, eids[gsl],
                             jnp.full((LANES,), -1, jnp.int32))
            gidx[...] = ids
            for j in range(EMBED_DIM):
                jv = jnp.full((LANES,), j, jnp.int32)
                vv = plsc.load_gather(buf, [parv, jv, c_loc])
                plsc.store_scatter(rowstage, [lane, jv], vv)
            pltpu.async_copy(
                rowstage,
                stage_hbm.at[plsc.Indices(gidx, ignored_value=-1)],
                esem).wait()
            return carry2

        ngroups = lax.div(seg_hi - (ebase0 + p0) + (LANES - 1), LANES)
        lax.fori_loop(0, ngroups, group_body, 0)

    fire(0)
    lax.fori_loop(0, nech, edrain, 0)

    def stripe_loop(t, carry):
        s = s0 + t

        @pl.when(t + 1 < STRIPES_PER_W)
        def _():
            fire(t + 1)

        drain(t)
        par = lax.rem(t, 2)

        @pl.when(s < N_STRIPES - 1)
        def _():
            do_stripe(s, par, STRIPE_W)

        @pl.when(s == N_STRIPES - 1)
        def _():
            do_stripe(s, par, LAST_W)

        return carry

    lax.fori_loop(0, STRIPES_PER_W, stripe_loop, 0)


DCHUNK = 128


def _dot_body(vstage_hbm, ustage_hbm, out_hbm, vrows, urows, res_v, sem):
    wid = lax.axis_index("s") * NUM_CORES + lax.axis_index("c")
    base = wid * B_PER_W

    lane = lax.iota(jnp.int32, LANES)

    def chunk_body(k, carry):
        rbase = base + k * DCHUNK
        cp1 = pltpu.async_copy(vstage_hbm.at[pl.ds(rbase, DCHUNK)], vrows,
                               sem)
        cp2 = pltpu.async_copy(ustage_hbm.at[pl.ds(rbase, DCHUNK)], urows,
                               sem)
        cp1.wait()
        cp2.wait()
        for g in range(DCHUNK // LANES):
            slot = g * LANES + lane
            acc = jnp.zeros((LANES,), jnp.float32)
            for j in range(EMBED_DIM):
                jv = jnp.full((LANES,), j, jnp.int32)
                vv = plsc.load_gather(vrows, [slot, jv])
                uu = plsc.load_gather(urows, [slot, jv])
                acc = acc + vv * uu
            res_v[pl.ds(k * DCHUNK + g * LANES, LANES)] = acc
        return carry

    lax.fori_loop(0, B_PER_W // DCHUNK, chunk_body, 0)

    pltpu.sync_copy(res_v, out_hbm.at[pl.ds(base, B_PER_W)])


def _make_mesh():
    return plsc.VectorSubcoreMesh(
        core_axis_name="c", subcore_axis_name="s",
        num_cores=NUM_CORES, num_subcores=NUM_SUBCORES)


def _scan_call(tbl, tail, svals, sids, offs):
    k = pl.kernel(
        _scan_body,
        out_type=jax.ShapeDtypeStruct((BATCH, WIDE), jnp.float32),
        mesh=_make_mesh(),
        compiler_params=pltpu.CompilerParams(needs_layout_passes=False),
        scratch_types=[
            pltpu.VMEM((2, EMBED_DIM, STRIPE_W), jnp.float32),
            pltpu.VMEM((EVCAP,), jnp.int32),
            pltpu.VMEM((EVCAP,), jnp.int32),
            pltpu.VMEM((LANES, WIDE), jnp.float32),
            pltpu.VMEM((LANES,), jnp.int32),
            pltpu.VMEM((OFFS_PAD,), jnp.int32),
            pltpu.SemaphoreType.DMA((2,)),
            pltpu.SemaphoreType.DMA,
        ],
    )
    return k(tbl, tail, svals, sids, offs)


def _dot_call(vstage, ustage):
    k = pl.kernel(
        _dot_body,
        out_type=jax.ShapeDtypeStruct((BATCH,), jnp.float32),
        mesh=_make_mesh(),
        compiler_params=pltpu.CompilerParams(needs_layout_passes=False),
        scratch_types=[
            pltpu.VMEM((DCHUNK, WIDE), jnp.float32),
            pltpu.VMEM((DCHUNK, WIDE), jnp.float32),
            pltpu.VMEM((B_PER_W,), jnp.float32),
            pltpu.SemaphoreType.DMA,
        ],
    )
    return k(vstage, ustage)


def _prep(idx):
    idx = idx.astype(jnp.int32)
    stripe = idx // STRIPE_W
    packed = stripe * BATCH + lax.iota(jnp.int32, BATCH)
    packed = jnp.sort(packed)
    sids = packed % BATCH
    svals = jnp.take(idx, sids, axis=0)
    sstripes = packed // BATCH
    bounds = lax.iota(jnp.int32, N_STRIPES + 1)
    offs = jnp.searchsorted(sstripes, bounds, side="left").astype(jnp.int32)
    offs = jnp.pad(offs, (0, OFFS_PAD - offs.shape[0]))
    svals = jnp.pad(svals, (0, ECHUNK))
    sids = jnp.pad(sids, (0, ECHUNK), constant_values=-1)
    return svals, sids, offs


def _tail_view(tbl):
    t = tbl[VOCAB - LAST_W:].T
    return jnp.pad(t, ((0, 0), (0, 128 - LAST_W)))


@jax.jit
def kernel(center_ids, context_ids, in_embed, out_embed):
    csv, csi, cof = _prep(center_ids)
    xsv, xsi, xof = _prep(context_ids)
    tin, ttin = in_embed.T, _tail_view(in_embed)
    tout, ttout = out_embed.T, _tail_view(out_embed)
    vstage = _scan_call(tin, ttin, csv, csi, cof)
    ustage = _scan_call(tout, ttout, xsv, xsi, xof)
    return _dot_call(vstage, ustage)
